# baseline (speedup 1.0000x reference)
import dataclasses

import jax
import jax.numpy as jnp
from jax.experimental import pallas as pl
from jax.experimental.pallas import tpu as pltpu
from jax.experimental.pallas import tpu_sc as plsc


def _sc_compiler_params():
    cp = pltpu.CompilerParams()
    if "needs_layout_passes" in pltpu.CompilerParams.__dataclass_fields__:
        cp = dataclasses.replace(cp, needs_layout_passes=False)
    return cp


_ROWS_PER_TILE = 2048


def _tc_argmin_body(z_ref, e_ref, idx_ref):
    z = z_ref[...]
    e = e_ref[...]
    k = e.shape[0]
    zt = z.T
    zn = jnp.sum(zt * zt, axis=0, keepdims=True)
    en = jnp.sum(e * e, axis=1, keepdims=True)
    p2 = jax.lax.dot_general(
        e * (-2.0), z, (((1,), (1,)), ((), ())),
        preferred_element_type=jnp.float32,
    )
    sub = jax.lax.broadcasted_iota(jnp.int32, (8, p2.shape[1]), 0)
    best_v = (zn + p2[0:8, :]) + en[0:8]
    best_i = sub
    for c in range(1, k // 8):
        vs = (zn + p2[8 * c:8 * (c + 1), :]) + en[8 * c:8 * (c + 1)]
        m = vs < best_v
        best_v = jnp.where(m, vs, best_v)
        best_i = jnp.where(m, sub + (8 * c), best_i)
    minv = jnp.min(best_v, axis=0, keepdims=True)
    sel = jnp.where(best_v == minv, best_i, jnp.int32(2**30))
    idx = jnp.min(sel, axis=0)
    idx_ref[0, 0, :] = idx


def _argmin_indices(z_flat, embedding):
    n, d = z_flat.shape
    k = embedding.shape[0]
    r = _ROWS_PER_TILE
    t = n // r
    out = pl.pallas_call(
        _tc_argmin_body,
        grid=(t,),
        in_specs=[
            pl.BlockSpec((r, d), lambda i: (i, 0)),
            pl.BlockSpec((k, d), lambda i: (0, 0)),
        ],
        out_specs=pl.BlockSpec((1, 1, r), lambda i: (i, 0, 0)),
        out_shape=jax.ShapeDtypeStruct((t, 1, r), jnp.int32),
    )(z_flat, embedding)
    return out.reshape(n)


_GATHER_CHUNK = 256
_SC_WORKERS = 32


def _sc_gather(embedding, indices):
    n = indices.shape[0]
    k, d = embedding.shape
    e_flat = embedding.reshape(k * d)
    per_w = n // _SC_WORKERS
    w = _GATHER_CHUNK
    nsteps = per_w // w
    mesh = plsc.VectorSubcoreMesh(core_axis_name="core",
                                  subcore_axis_name="subcore")

    @pl.kernel(out_type=jax.ShapeDtypeStruct((n * d,), embedding.dtype),
               mesh=mesh,
               compiler_params=_sc_compiler_params(),
               scratch_types=[
                   pltpu.VMEM((k * d,), jnp.float32),
                   pltpu.VMEM((per_w,), jnp.int32),
                   pltpu.VMEM((w * d,), jnp.float32),
                   pltpu.VMEM((w * d,), jnp.float32),
                   pltpu.SemaphoreType.DMA,
                   pltpu.SemaphoreType.DMA,
               ])
    def gather_kernel(e_hbm, i_hbm, o_hbm, table_v, idx_v, out_a, out_b,
                      sem_a, sem_b):
        core = jax.lax.axis_index("core")
        sub = jax.lax.axis_index("subcore")
        base = (sub * 2 + core) * per_w
        pltpu.sync_copy(i_hbm.at[pl.ds(base, per_w)], idx_v)
        pltpu.sync_copy(e_hbm, table_v)
        bufs = (out_a, out_b)
        sems = (sem_a, sem_b)
        lane = jax.lax.broadcasted_iota(jnp.int32, (16,), 0)
        copies = [None] * nsteps
        for g in range(nsteps):
            buf = bufs[g % 2]
            if g >= 2:
                copies[g - 2].wait()

            @pl.loop(0, w)
            def _(r):
                idv = plsc.load_gather(idx_v, [jnp.full((16,), g * w + r,
                                                        jnp.int32)])
                addr = idv * d + lane
                for c in range(d // 16):
                    val = plsc.load_gather(table_v, [addr + (16 * c)])
                    buf[pl.ds(r * d + 16 * c, 16)] = val

            copies[g] = pltpu.async_copy(
                buf, o_hbm.at[pl.ds((base + g * w) * d, w * d)], sems[g % 2])
        for g in range(max(0, nsteps - 2), nsteps):
            copies[g].wait()

    return gather_kernel(e_flat, indices).reshape(n, d)


def kernel(z, embedding):
    d = embedding.shape[1]
    z_flat = z.reshape(-1, d)
    indices = _argmin_indices(z_flat, embedding)
    z_q = _sc_gather(embedding, indices)
    return z_q.reshape(z.shape), indices


# speedup vs baseline: 1.0033x; 1.0033x over previous
import dataclasses

import jax
import jax.numpy as jnp
from jax.experimental import pallas as pl
from jax.experimental.pallas import tpu as pltpu
from jax.experimental.pallas import tpu_sc as plsc


def _sc_compiler_params():
    cp = pltpu.CompilerParams()
    if "needs_layout_passes" in pltpu.CompilerParams.__dataclass_fields__:
        cp = dataclasses.replace(cp, needs_layout_passes=False)
    return cp


_ROWS_PER_TILE = 2048


def _tc_argmin_body(z_ref, e_ref, idx_ref):
    z = z_ref[...]
    e = e_ref[...]
    k = e.shape[0]
    zt = z.T
    zn = jnp.sum(zt * zt, axis=0, keepdims=True)
    en = jnp.sum(e * e, axis=1, keepdims=True)
    p2 = jax.lax.dot_general(
        e * (-2.0), z, (((1,), (1,)), ((), ())),
        preferred_element_type=jnp.float32,
    )
    sub = jax.lax.broadcasted_iota(jnp.int32, (8, p2.shape[1]), 0)
    best_v = (zn + p2[0:8, :]) + en[0:8]
    best_i = sub
    for c in range(1, k // 8):
        vs = (zn + p2[8 * c:8 * (c + 1), :]) + en[8 * c:8 * (c + 1)]
        m = vs < best_v
        best_v = jnp.where(m, vs, best_v)
        best_i = jnp.where(m, sub + (8 * c), best_i)
    minv = jnp.min(best_v, axis=0, keepdims=True)
    sel = jnp.where(best_v == minv, best_i, jnp.int32(2**30))
    idx = jnp.min(sel, axis=0)
    idx_ref[0, 0, :] = idx


def _argmin_indices(z_flat, embedding):
    n, d = z_flat.shape
    k = embedding.shape[0]
    r = _ROWS_PER_TILE
    t = n // r
    out = pl.pallas_call(
        _tc_argmin_body,
        grid=(t,),
        in_specs=[
            pl.BlockSpec((r, d), lambda i: (i, 0)),
            pl.BlockSpec((k, d), lambda i: (0, 0)),
        ],
        out_specs=pl.BlockSpec((1, 1, r), lambda i: (i, 0, 0)),
        out_shape=jax.ShapeDtypeStruct((t, 1, r), jnp.int32),
    )(z_flat, embedding)
    return out.reshape(n)


_GATHER_CHUNK = 128
_SC_WORKERS = 32


def _sc_gather(embedding, indices):
    n = indices.shape[0]
    k, d = embedding.shape
    e2 = embedding.reshape(k // 2, 2 * d)
    idx2 = indices.reshape(n // 128, 128)
    per_w = n // _SC_WORKERS
    w = _GATHER_CHUNK
    nsteps = per_w // w
    mesh = plsc.VectorSubcoreMesh(core_axis_name="core",
                                  subcore_axis_name="subcore")

    @pl.kernel(out_type=jax.ShapeDtypeStruct((n // 2, 2 * d), embedding.dtype),
               mesh=mesh,
               compiler_params=_sc_compiler_params(),
               scratch_types=[
                   pltpu.VMEM((k // 2, 2 * d), jnp.float32),
                   pltpu.VMEM((per_w // 128, 128), jnp.int32),
                   pltpu.VMEM((w // 2, 2 * d), jnp.float32),
                   pltpu.VMEM((w // 2, 2 * d), jnp.float32),
                   pltpu.SemaphoreType.DMA,
                   pltpu.SemaphoreType.DMA,
               ])
    def gather_kernel(e_hbm, i_hbm, o_hbm, table_v, idx_v, out_a, out_b,
                      sem_a, sem_b):
        core = jax.lax.axis_index("core")
        sub = jax.lax.axis_index("subcore")
        wid = sub * 2 + core
        pltpu.sync_copy(i_hbm.at[pl.ds(wid * (per_w // 128), per_w // 128)],
                        idx_v)
        pltpu.sync_copy(e_hbm, table_v)
        bufs = (out_a, out_b)
        sems = (sem_a, sem_b)
        lane = jax.lax.broadcasted_iota(jnp.int32, (16,), 0)
        copies = [None] * nsteps
        for g in range(nsteps):
            buf = bufs[g % 2]
            if g >= 2:
                copies[g - 2].wait()
            g_vec = jnp.full((16,), g, jnp.int32)

            @pl.loop(0, w, step=16)
            def _(r):
                for j in range(16):
                    idv = plsc.load_gather(
                        idx_v, [g_vec, jnp.full((16,), r + j, jnp.int32)])
                    rowv = idv >> 1
                    colv = ((idv & 1) << 6) + lane
                    rdst = (r >> 1) + (j >> 1)
                    cdst = (j & 1) * d
                    for c in range(d // 16):
                        val = plsc.load_gather(table_v,
                                               [rowv, colv + (16 * c)])
                        buf[rdst, pl.ds(cdst + 16 * c, 16)] = val

            copies[g] = pltpu.async_copy(
                buf, o_hbm.at[pl.ds(wid * (per_w // 2) + g * (w // 2), w // 2)],
                sems[g % 2])
        for g in range(max(0, nsteps - 2), nsteps):
            copies[g].wait()

    return gather_kernel(e2, idx2).reshape(n, d)


def kernel(z, embedding):
    d = embedding.shape[1]
    z_flat = z.reshape(-1, d)
    indices = _argmin_indices(z_flat, embedding)
    z_q = _sc_gather(embedding, indices)
    return z_q.reshape(z.shape), indices
